# Initial kernel scaffold; baseline (speedup 1.0000x reference)
import jax
import jax.numpy as jnp
from jax.experimental import pallas as pl


def kernel(x, edge_index, W_msg, W_self, b):
    raise NotImplementedError("write your pallas kernel here")


# speedup vs baseline: 5.1565x; 5.1565x over previous
import jax
import jax.numpy as jnp
from jax import lax
from jax.experimental import pallas as pl
from jax.experimental.pallas import tpu as pltpu
from jax.experimental.pallas import tpu_sc as plsc

N = 10000
E = 160000
D = 256
DH = 128
NC = 2
NS = 16
B = 128
K = 80
EP = NS * K * B
NP = N + 128


def _sc_feat_body(xt, srcs, dsts, zacc, s2, acc_sh, src_v, dst_v, buf, sem_g, sem_s):
    c = lax.axis_index("c")
    s = lax.axis_index("s")

    @pl.when(s == 0)
    def _():
        pltpu.sync_copy(zacc, acc_sh)

    pltpu.sync_copy(srcs.at[s], src_v)
    pltpu.sync_copy(dsts.at[s], dst_v)
    plsc.subcore_barrier()

    def chunk(j, carry):
        pltpu.async_copy(xt.at[c].at[src_v.at[j]], buf, sem_g).wait()
        pltpu.async_copy(buf, acc_sh.at[dst_v.at[j]], sem_s, add=True).wait()
        return carry

    lax.fori_loop(0, K, chunk, 0)
    plsc.subcore_barrier()

    @pl.when(s == 0)
    def _():
        pltpu.sync_copy(acc_sh, s2.at[c])


def _sc_deg_body(dsts, zdeg, ones_h, d2, deg_sh, dst_v, ones_v, sem_d):
    c = lax.axis_index("c")
    s = lax.axis_index("s")

    @pl.when(s == 0)
    def _():
        pltpu.sync_copy(zdeg, deg_sh)

    pltpu.sync_copy(dsts.at[s], dst_v)
    pltpu.sync_copy(ones_h, ones_v)
    plsc.subcore_barrier()

    def chunk(j, carry):
        @pl.when(lax.rem(j, 2) == c)
        def _():
            pltpu.async_copy(ones_v, deg_sh.at[dst_v.at[j]], sem_d, add=True).wait()
        return carry

    lax.fori_loop(0, K, chunk, 0)
    plsc.subcore_barrier()

    @pl.when(s == 0)
    def _():
        pltpu.sync_copy(deg_sh, d2.at[c])


def _sc_aggregate(xt, srcs, dsts):
    mesh = plsc.VectorSubcoreMesh(core_axis_name="c", subcore_axis_name="s",
                                  num_cores=NC, num_subcores=NS)
    feat = pl.kernel(
        _sc_feat_body,
        out_type=jax.ShapeDtypeStruct((NC, NP, DH), jnp.float32),
        mesh=mesh,
        scratch_types=[
            pltpu.VMEM_SHARED((NP, DH), jnp.float32),
            pltpu.VMEM((K, B), jnp.int32),
            pltpu.VMEM((K, B), jnp.int32),
            pltpu.VMEM((B, DH), jnp.float32),
            pltpu.SemaphoreType.DMA,
            pltpu.SemaphoreType.DMA,
        ],
    )
    deg = pl.kernel(
        _sc_deg_body,
        out_type=jax.ShapeDtypeStruct((NC, NP, DH), jnp.float32),
        mesh=mesh,
        scratch_types=[
            pltpu.VMEM_SHARED((NP, DH), jnp.float32),
            pltpu.VMEM((K, B), jnp.int32),
            pltpu.VMEM((B, DH), jnp.float32),
            pltpu.SemaphoreType.DMA,
        ],
    )
    zacc = jnp.zeros((NP, DH), jnp.float32)
    zdeg = jnp.zeros((NP, DH), jnp.float32)
    ones_h = jnp.ones((B, DH), jnp.float32)
    s2 = feat(xt, srcs, dsts, zacc)
    d2 = deg(dsts, zdeg, ones_h)
    return s2, d2


def _tc_body(s0, s1, d2, x, wm0, wm1, ws, bb, out):
    deg = jnp.maximum(d2[0] + d2[1], 1.0)
    agg = (jnp.dot(s0[...], wm0[...], preferred_element_type=jnp.float32)
           + jnp.dot(s1[...], wm1[...], preferred_element_type=jnp.float32))
    self_path = jnp.dot(x[...], ws[...], preferred_element_type=jnp.float32)
    out[...] = jnp.maximum(agg / deg + self_path + bb[0:1, :], 0.0)


def _tc_combine(s2, d2, x, w_msg, w_self, b):
    R = 1000
    bb = jnp.broadcast_to(b, (8, D))
    return pl.pallas_call(
        _tc_body,
        grid=(N // R,),
        in_specs=[
            pl.BlockSpec((R, DH), lambda i: (i, 0)),
            pl.BlockSpec((R, DH), lambda i: (i, 0)),
            pl.BlockSpec((NC, R, 1), lambda i: (0, i, 0)),
            pl.BlockSpec((R, D), lambda i: (i, 0)),
            pl.BlockSpec((DH, D), lambda i: (0, 0)),
            pl.BlockSpec((DH, D), lambda i: (0, 0)),
            pl.BlockSpec((D, D), lambda i: (0, 0)),
            pl.BlockSpec((8, D), lambda i: (0, 0)),
        ],
        out_specs=pl.BlockSpec((R, D), lambda i: (i, 0)),
        out_shape=jax.ShapeDtypeStruct((N, D), jnp.float32),
    )(s2[0], s2[1], d2[:, :N, 0:1], x, w_msg[:DH], w_msg[DH:], w_self, bb)


def kernel(x, edge_index, W_msg, W_self, b):
    xt = x.reshape(N, NC, DH).transpose(1, 0, 2)
    pad = EP - E
    pad_src = (jnp.arange(pad, dtype=jnp.int32) * 97) % N
    pad_dst = N + (jnp.arange(pad, dtype=jnp.int32) % (NP - N))
    srcs = jnp.concatenate([edge_index[0], pad_src]).reshape(NS, K, B)
    dsts = jnp.concatenate([edge_index[1], pad_dst]).reshape(NS, K, B)
    s2, d2 = _sc_aggregate(xt, srcs, dsts)
    return _tc_combine(s2, d2, x, W_msg, W_self, b)


# speedup vs baseline: 6.6395x; 1.2876x over previous
import jax
import jax.numpy as jnp
from jax import lax
from jax.experimental import pallas as pl
from jax.experimental.pallas import tpu as pltpu
from jax.experimental.pallas import tpu_sc as plsc

N = 10000
E = 160000
D = 256
DH = 128
NC = 2
NS = 16
B = 128
K = 80
KH = 40
EP = NS * K * B
NP = N + 128


def _sc_feat_body(xt, srcs, dsts, zacc, s2, acc_sh, src_v, dst_v, buf0, buf1,
                  sg0, sg1, sem_s):
    c = lax.axis_index("c")
    s = lax.axis_index("s")

    @pl.when(s == 0)
    def _():
        pltpu.sync_copy(zacc, acc_sh)

    plsc.subcore_barrier()

    def gather(j, buf, sem):
        pltpu.async_copy(xt.at[c].at[src_v.at[j]], buf, sem)

    def gwait(buf, sem):
        pltpu.make_async_copy(xt.at[c, pl.ds(0, B)], buf, sem).wait()

    def scat(j, buf):
        pltpu.async_copy(buf, acc_sh.at[dst_v.at[j]], sem_s, add=True).wait()

    for h in range(K // KH):
        pltpu.sync_copy(srcs.at[s, pl.ds(h * KH, KH)], src_v)
        pltpu.sync_copy(dsts.at[s, pl.ds(h * KH, KH)], dst_v)
        gather(0, buf0, sg0)

        def pair(g, carry):
            j0 = 2 * g
            gather(j0 + 1, buf1, sg1)
            gwait(buf0, sg0)
            scat(j0, buf0)
            gather(jnp.minimum(j0 + 2, KH - 1), buf0, sg0)
            gwait(buf1, sg1)
            scat(j0 + 1, buf1)
            return carry

        lax.fori_loop(0, KH // 2, pair, 0)
        gwait(buf0, sg0)
    plsc.subcore_barrier()

    @pl.when(s == 0)
    def _():
        pltpu.sync_copy(acc_sh, s2.at[c])


def _sc_deg_body(dsts, zdeg, ones_h, d2, deg_sh, dst_v, ones_v, sem_d):
    c = lax.axis_index("c")
    s = lax.axis_index("s")

    @pl.when(s == 0)
    def _():
        pltpu.sync_copy(zdeg, deg_sh)

    pltpu.sync_copy(dsts.at[s], dst_v)
    pltpu.sync_copy(ones_h, ones_v)
    plsc.subcore_barrier()

    def chunk(j, carry):
        @pl.when(lax.rem(j, 2) == c)
        def _():
            pltpu.async_copy(ones_v, deg_sh.at[dst_v.at[j]], sem_d, add=True).wait()
        return carry

    lax.fori_loop(0, K, chunk, 0)
    plsc.subcore_barrier()

    @pl.when(s == 0)
    def _():
        pltpu.sync_copy(deg_sh, d2.at[c])


def _sc_aggregate(xt, srcs, dsts):
    mesh = plsc.VectorSubcoreMesh(core_axis_name="c", subcore_axis_name="s",
                                  num_cores=NC, num_subcores=NS)
    feat = pl.kernel(
        _sc_feat_body,
        out_type=jax.ShapeDtypeStruct((NC, NP, DH), jnp.float32),
        mesh=mesh,
        scratch_types=[
            pltpu.VMEM_SHARED((NP, DH), jnp.float32),
            pltpu.VMEM((KH, B), jnp.int32),
            pltpu.VMEM((KH, B), jnp.int32),
            pltpu.VMEM((B, DH), jnp.float32),
            pltpu.VMEM((B, DH), jnp.float32),
            pltpu.SemaphoreType.DMA,
            pltpu.SemaphoreType.DMA,
            pltpu.SemaphoreType.DMA,
        ],
    )
    deg = pl.kernel(
        _sc_deg_body,
        out_type=jax.ShapeDtypeStruct((NC, NP, DH), jnp.float32),
        mesh=mesh,
        scratch_types=[
            pltpu.VMEM_SHARED((NP, DH), jnp.float32),
            pltpu.VMEM((K, B), jnp.int32),
            pltpu.VMEM((B, DH), jnp.float32),
            pltpu.SemaphoreType.DMA,
        ],
    )
    zacc = jnp.zeros((NP, DH), jnp.float32)
    zdeg = jnp.zeros((NP, DH), jnp.float32)
    ones_h = jnp.ones((B, DH), jnp.float32)
    s2 = feat(xt, srcs, dsts, zacc)
    d2 = deg(dsts, zdeg, ones_h)
    return s2, d2


def _tc_body(s0, s1, d2, x, wm0, wm1, ws, bb, out):
    deg = jnp.maximum(d2[0] + d2[1], 1.0)
    agg = (jnp.dot(s0[...], wm0[...], preferred_element_type=jnp.float32)
           + jnp.dot(s1[...], wm1[...], preferred_element_type=jnp.float32))
    self_path = jnp.dot(x[...], ws[...], preferred_element_type=jnp.float32)
    out[...] = jnp.maximum(agg / deg + self_path + bb[0:1, :], 0.0)


def _tc_combine(s2, d2, x, w_msg, w_self, b):
    R = 1000
    bb = jnp.broadcast_to(b, (8, D))
    return pl.pallas_call(
        _tc_body,
        grid=(N // R,),
        in_specs=[
            pl.BlockSpec((R, DH), lambda i: (i, 0)),
            pl.BlockSpec((R, DH), lambda i: (i, 0)),
            pl.BlockSpec((NC, R, 1), lambda i: (0, i, 0)),
            pl.BlockSpec((R, D), lambda i: (i, 0)),
            pl.BlockSpec((DH, D), lambda i: (0, 0)),
            pl.BlockSpec((DH, D), lambda i: (0, 0)),
            pl.BlockSpec((D, D), lambda i: (0, 0)),
            pl.BlockSpec((8, D), lambda i: (0, 0)),
        ],
        out_specs=pl.BlockSpec((R, D), lambda i: (i, 0)),
        out_shape=jax.ShapeDtypeStruct((N, D), jnp.float32),
    )(s2[0], s2[1], d2[:, :N, 0:1], x, w_msg[:DH], w_msg[DH:], w_self, bb)


def kernel(x, edge_index, W_msg, W_self, b):
    xt = x.reshape(N, NC, DH).transpose(1, 0, 2)
    pad = EP - E
    pad_src = (jnp.arange(pad, dtype=jnp.int32) * 97) % N
    pad_dst = N + (jnp.arange(pad, dtype=jnp.int32) % (NP - N))
    srcs = jnp.concatenate([edge_index[0], pad_src]).reshape(NS, K, B)
    dsts = jnp.concatenate([edge_index[1], pad_dst]).reshape(NS, K, B)
    s2, d2 = _sc_aggregate(xt, srcs, dsts)
    return _tc_combine(s2, d2, x, W_msg, W_self, b)


# speedup vs baseline: 8.0307x; 1.2095x over previous
import jax
import jax.numpy as jnp
from jax import lax
from jax.experimental import pallas as pl
from jax.experimental.pallas import tpu as pltpu
from jax.experimental.pallas import tpu_sc as plsc

N = 10000
E = 160000
D = 256
DH = 128
NC = 2
NS = 16
B = 128
K = 80
KH = 40
EP = NS * K * B
NP = N + 128


def _sc_feat_body(xt, srcs, dsts, zacc, s2, acc_sh, src_v, dst_v, buf0, buf1,
                  sg0, sg1, sem_s):
    c = lax.axis_index("c")
    s = lax.axis_index("s")

    @pl.when(s == 0)
    def _():
        pltpu.sync_copy(zacc, acc_sh)

    plsc.subcore_barrier()

    def gather(j, buf, sem):
        pltpu.async_copy(xt.at[c].at[src_v.at[j]], buf, sem)

    def gwait(buf, sem):
        pltpu.make_async_copy(xt.at[c, pl.ds(0, B)], buf, sem).wait()

    def scat(j, buf):
        pltpu.async_copy(buf, acc_sh.at[dst_v.at[j]], sem_s, add=True).wait()

    for h in range(K // KH):
        pltpu.sync_copy(srcs.at[s, pl.ds(h * KH, KH)], src_v)
        pltpu.sync_copy(dsts.at[s, pl.ds(h * KH, KH)], dst_v)
        gather(0, buf0, sg0)

        def pair(g, carry):
            j0 = 2 * g
            gather(j0 + 1, buf1, sg1)
            gwait(buf0, sg0)
            scat(j0, buf0)
            gather(jnp.minimum(j0 + 2, KH - 1), buf0, sg0)
            gwait(buf1, sg1)
            scat(j0 + 1, buf1)
            return carry

        lax.fori_loop(0, KH // 2, pair, 0)
        gwait(buf0, sg0)
    plsc.subcore_barrier()

    @pl.when(s == 0)
    def _():
        pltpu.sync_copy(acc_sh, s2.at[c])


def _sc_deg_body(dsts, d2, dst_v, cnt):
    c = lax.axis_index("c")
    s = lax.axis_index("s")

    pltpu.sync_copy(dsts.at[s], dst_v)

    zeros16 = jnp.zeros((16,), jnp.float32)
    ones16 = jnp.ones((16,), jnp.float32)

    def zero(i, carry):
        cnt[pl.ds(i * 16, 16)] = zeros16
        return carry

    lax.fori_loop(0, NP // 16, zero, 0)

    def chunk(g, carry):
        j = 2 * g + c
        for v in range(B // 16):
            idx = dst_v[pl.ds(j * B + v * 16, 16)]
            plsc.addupdate_scatter(cnt, [idx], ones16)
        return carry

    lax.fori_loop(0, K // 2, chunk, 0)
    pltpu.sync_copy(cnt, d2.at[c, s])


def _sc_aggregate(xt, srcs, dsts):
    mesh = plsc.VectorSubcoreMesh(core_axis_name="c", subcore_axis_name="s",
                                  num_cores=NC, num_subcores=NS)
    feat = pl.kernel(
        _sc_feat_body,
        out_type=jax.ShapeDtypeStruct((NC, NP, DH), jnp.float32),
        mesh=mesh,
        scratch_types=[
            pltpu.VMEM_SHARED((NP, DH), jnp.float32),
            pltpu.VMEM((KH, B), jnp.int32),
            pltpu.VMEM((KH, B), jnp.int32),
            pltpu.VMEM((B, DH), jnp.float32),
            pltpu.VMEM((B, DH), jnp.float32),
            pltpu.SemaphoreType.DMA,
            pltpu.SemaphoreType.DMA,
            pltpu.SemaphoreType.DMA,
        ],
    )
    deg = pl.kernel(
        _sc_deg_body,
        out_type=jax.ShapeDtypeStruct((NC, NS, NP), jnp.float32),
        mesh=mesh,
        compiler_params=pltpu.CompilerParams(needs_layout_passes=False),
        scratch_types=[
            pltpu.VMEM((K * B,), jnp.int32),
            pltpu.VMEM((NP,), jnp.float32),
        ],
    )
    zacc = jnp.zeros((NP, DH), jnp.float32)
    s2 = feat(xt, srcs, dsts, zacc)
    d2 = deg(dsts.reshape(NS, K * B))
    return s2, d2


def _tc_body(s0, s1, d2, x, wm0, wm1, ws, bb, out):
    deg = jnp.maximum(jnp.sum(d2[...], axis=1, keepdims=True), 1.0)
    agg = (jnp.dot(s0[...], wm0[...], preferred_element_type=jnp.float32)
           + jnp.dot(s1[...], wm1[...], preferred_element_type=jnp.float32))
    self_path = jnp.dot(x[...], ws[...], preferred_element_type=jnp.float32)
    out[...] = jnp.maximum(agg / deg + self_path + bb[0:1, :], 0.0)


def _tc_combine(s2, d2, x, w_msg, w_self, b):
    R = 1000
    bb = jnp.broadcast_to(b, (8, D))
    return pl.pallas_call(
        _tc_body,
        grid=(N // R,),
        in_specs=[
            pl.BlockSpec((R, DH), lambda i: (i, 0)),
            pl.BlockSpec((R, DH), lambda i: (i, 0)),
            pl.BlockSpec((R, NC * NS), lambda i: (i, 0)),
            pl.BlockSpec((R, D), lambda i: (i, 0)),
            pl.BlockSpec((DH, D), lambda i: (0, 0)),
            pl.BlockSpec((DH, D), lambda i: (0, 0)),
            pl.BlockSpec((D, D), lambda i: (0, 0)),
            pl.BlockSpec((8, D), lambda i: (0, 0)),
        ],
        out_specs=pl.BlockSpec((R, D), lambda i: (i, 0)),
        out_shape=jax.ShapeDtypeStruct((N, D), jnp.float32),
    )(s2[0], s2[1], d2.reshape(NC * NS, NP).T, x, w_msg[:DH], w_msg[DH:], w_self, bb)


def kernel(x, edge_index, W_msg, W_self, b):
    xt = x.reshape(N, NC, DH).transpose(1, 0, 2)
    pad = EP - E
    pad_src = (jnp.arange(pad, dtype=jnp.int32) * 97) % N
    pad_dst = N + (jnp.arange(pad, dtype=jnp.int32) % (NP - N))
    srcs = jnp.concatenate([edge_index[0], pad_src]).reshape(NS, K, B)
    dsts = jnp.concatenate([edge_index[1], pad_dst]).reshape(NS, K, B)
    s2, d2 = _sc_aggregate(xt, srcs, dsts)
    return _tc_combine(s2, d2, x, W_msg, W_self, b)
